# baseline (speedup 1.0000x reference)
import functools

import jax
import jax.numpy as jnp
from jax import lax
from jax.experimental import pallas as pl
from jax.experimental.pallas import tpu as pltpu
from jax.experimental.pallas import tpu_sc as plsc

ID_NUM = 100000
BATCH = 16384
DIM = 32

NC = 2
NS = 16
L = 16
NW = NC * NS
CHUNK = 128
NCHUNK = BATCH // CHUNK

DGRP_STRIDE = 6250 * 8 * 128
WINDOW = 6399104


def _paddr_body(exp_ref, id_ref, o_ref):
    p = exp_ref[...] * ID_NUM + id_ref[...]
    o_ref[...] = ((p >> 7) << 10) + (p & 127)


_paddr_call = pl.pallas_call(
    _paddr_body,
    out_shape=jax.ShapeDtypeStruct((CHUNK, CHUNK), jnp.int32),
)


@functools.partial(
    pl.kernel,
    out_type=jax.ShapeDtypeStruct((4, NCHUNK, 8, CHUNK), jnp.float32),
    mesh=plsc.VectorSubcoreMesh(core_axis_name="c", subcore_axis_name="s"),
    scratch_types=[
        pltpu.VMEM((BATCH,), jnp.int32),
        pltpu.VMEM((1, NCHUNK, 1, CHUNK), jnp.float32),
        [pltpu.SemaphoreType.DMA] * 8,
        pltpu.SemaphoreType.DMA,
    ],
    compiler_params=pltpu.CompilerParams(use_tc_tiling_on_sc=False),
)
def _gather_kernel(paddr_hbm, flat_hbm, out_hbm, paddr_v, vals_v, gsems, sem_o):
    d = lax.axis_index("s") * NC + lax.axis_index("c")
    a = d // 8
    j = d - a * 8
    base = a * DGRP_STRIDE + j * CHUNK
    ngrp = len(gsems)
    gch = NCHUNK // ngrp

    pltpu.sync_copy(paddr_hbm, paddr_v)
    window = flat_hbm.at[pl.ds(base, WINDOW)]

    for g in range(ngrp):
        sg = gsems[g]

        @pl.loop(g * gch, (g + 1) * gch)
        def _fire(c):
            pltpu.async_copy(window.at[paddr_v.at[pl.ds(c * CHUNK, CHUNK)]],
                             vals_v.at[0, c, 0, :], sg)

    for g in range(ngrp):
        drain = pltpu.make_async_copy(
            window.at[paddr_v.at[pl.ds(0, CHUNK)]], vals_v.at[0, 0, 0, :],
            gsems[g])

        @pl.loop(0, gch)
        def _drain(c):
            drain.wait()

        pltpu.async_copy(
            vals_v.at[pl.ds(0, 1), pl.ds(g * gch, gch), pl.ds(0, 1), :],
            out_hbm.at[pl.ds(a, 1), pl.ds(g * gch, gch), pl.ds(j, 1), :],
            sem_o)

    owait = pltpu.make_async_copy(
        vals_v.at[pl.ds(0, 1), pl.ds(0, gch), pl.ds(0, 1), :],
        out_hbm.at[pl.ds(0, 1), pl.ds(0, gch), pl.ds(0, 1), :], sem_o)

    @pl.loop(0, ngrp)
    def _owait(_):
        owait.wait()


def kernel(exp_infor, id_infor, detail_embeddings):
    flat = detail_embeddings.reshape(6250, 128, 4, 8)
    flat = flat.transpose(2, 0, 3, 1).reshape(-1)
    paddr = _paddr_call(exp_infor.reshape(CHUNK, CHUNK),
                        id_infor.reshape(CHUNK, CHUNK)).reshape(-1)
    out4d = _gather_kernel(paddr, flat)
    return out4d.transpose(1, 3, 0, 2).reshape(BATCH, DIM)


# speedup vs baseline: 1.0242x; 1.0242x over previous
import functools

import jax
import jax.numpy as jnp
from jax import lax
from jax.experimental import pallas as pl
from jax.experimental.pallas import tpu as pltpu
from jax.experimental.pallas import tpu_sc as plsc

ID_NUM = 100000
BATCH = 16384
DIM = 32

NC = 2
NS = 16
L = 16
NW = NC * NS
CHUNK = 128
NCHUNK = BATCH // CHUNK

DGRP_STRIDE = 6250 * 8 * 128
WINDOW = 6399104


def _paddr_body(exp_ref, id_ref, o_ref):
    p = exp_ref[...] * ID_NUM + id_ref[...]
    o_ref[...] = ((p >> 7) << 10) + (p & 127)


_paddr_call = pl.pallas_call(
    _paddr_body,
    out_shape=jax.ShapeDtypeStruct((CHUNK, CHUNK), jnp.int32),
)


@functools.partial(
    pl.kernel,
    out_type=jax.ShapeDtypeStruct((4, NCHUNK, 8, CHUNK), jnp.float32),
    mesh=plsc.VectorSubcoreMesh(core_axis_name="c", subcore_axis_name="s"),
    scratch_types=[
        pltpu.VMEM((BATCH,), jnp.int32),
        pltpu.VMEM((1, NCHUNK, 1, CHUNK), jnp.float32),
        pltpu.SemaphoreType.DMA,
    ],
    compiler_params=pltpu.CompilerParams(use_tc_tiling_on_sc=False, skip_device_barrier=True),
)
def _gather_kernel(paddr_hbm, flat_hbm, out_hbm, paddr_v, vals_v, sem):
    d = lax.axis_index("s") * NC + lax.axis_index("c")
    a = d // 8
    j = d - a * 8
    base = a * DGRP_STRIDE + j * CHUNK

    pltpu.sync_copy(paddr_hbm, paddr_v)
    window = flat_hbm.at[pl.ds(base, WINDOW)]

    @pl.loop(0, NCHUNK)
    def _fire(c):
        pltpu.async_copy(window.at[paddr_v.at[pl.ds(c * CHUNK, CHUNK)]],
                         vals_v.at[0, c, 0, :], sem)

    drain = pltpu.make_async_copy(
        window.at[paddr_v.at[pl.ds(0, CHUNK)]], vals_v.at[0, 0, 0, :], sem)

    @pl.loop(0, NCHUNK)
    def _drain(c):
        drain.wait()

    pltpu.sync_copy(vals_v,
                    out_hbm.at[pl.ds(a, 1), :, pl.ds(j, 1), :])


def kernel(exp_infor, id_infor, detail_embeddings):
    flat = detail_embeddings.reshape(6250, 128, 4, 8)
    flat = flat.transpose(2, 0, 3, 1).reshape(-1)
    paddr = _paddr_call(exp_infor.reshape(CHUNK, CHUNK),
                        id_infor.reshape(CHUNK, CHUNK)).reshape(-1)
    out4d = _gather_kernel(paddr, flat)
    return out4d.transpose(1, 3, 0, 2).reshape(BATCH, DIM)


# speedup vs baseline: 1.0345x; 1.0101x over previous
import functools

import jax
import jax.numpy as jnp
from jax import lax
from jax.experimental import pallas as pl
from jax.experimental.pallas import tpu as pltpu
from jax.experimental.pallas import tpu_sc as plsc

ID_NUM = 100000
BATCH = 16384
DIM = 32

NC = 2
NS = 16
L = 16
NW = NC * NS
CHUNK = 128
NCHUNK = BATCH // CHUNK

DGRP_STRIDE = 6250 * 8 * 128
WINDOW = 6399104


def _paddr_body(exp_ref, id_ref, o_ref):
    p = exp_ref[...] * ID_NUM + id_ref[...]
    o_ref[...] = ((p >> 7) << 10) + (p & 127)


_paddr_call = pl.pallas_call(
    _paddr_body,
    out_shape=jax.ShapeDtypeStruct((CHUNK, CHUNK), jnp.int32),
)


@functools.partial(
    pl.kernel,
    out_type=jax.ShapeDtypeStruct((4, NCHUNK, 8, CHUNK), jnp.float32),
    mesh=plsc.VectorSubcoreMesh(core_axis_name="c", subcore_axis_name="s"),
    scratch_types=[
        pltpu.VMEM((BATCH,), jnp.int32),
        pltpu.VMEM((1, NCHUNK, 1, CHUNK), jnp.float32),
        pltpu.SemaphoreType.DMA,
        pltpu.SemaphoreType.DMA,
        pltpu.SemaphoreType.DMA,
    ],
    compiler_params=pltpu.CompilerParams(use_tc_tiling_on_sc=False),
)
def _gather_kernel(paddr_hbm, flat_hbm, out_hbm, paddr_v, vals_v, sem, sem_a, sem_b):
    d = lax.axis_index("s") * NC + lax.axis_index("c")
    a = d // 8
    j = d - a * 8
    base = a * DGRP_STRIDE + j * CHUNK

    window = flat_hbm.at[pl.ds(base, WINDOW)]
    half = BATCH // 2
    st0 = pltpu.async_copy(paddr_hbm.at[pl.ds(0, half)],
                           paddr_v.at[pl.ds(0, half)], sem_a)
    st1 = pltpu.async_copy(paddr_hbm.at[pl.ds(half, half)],
                           paddr_v.at[pl.ds(half, half)], sem_b)
    st0.wait()

    @pl.loop(0, NCHUNK // 2)
    def _fire0(c):
        pltpu.async_copy(window.at[paddr_v.at[pl.ds(c * CHUNK, CHUNK)]],
                         vals_v.at[0, c, 0, :], sem)

    st1.wait()

    @pl.loop(NCHUNK // 2, NCHUNK)
    def _fire1(c):
        pltpu.async_copy(window.at[paddr_v.at[pl.ds(c * CHUNK, CHUNK)]],
                         vals_v.at[0, c, 0, :], sem)

    drain = pltpu.make_async_copy(
        window.at[paddr_v.at[pl.ds(0, CHUNK)]], vals_v.at[0, 0, 0, :], sem)

    @pl.loop(0, NCHUNK)
    def _drain(c):
        drain.wait()

    pltpu.sync_copy(vals_v,
                    out_hbm.at[pl.ds(a, 1), :, pl.ds(j, 1), :])


def kernel(exp_infor, id_infor, detail_embeddings):
    flat = detail_embeddings.reshape(6250, 128, 4, 8)
    flat = flat.transpose(2, 0, 3, 1).reshape(-1)
    paddr = _paddr_call(exp_infor.reshape(CHUNK, CHUNK),
                        id_infor.reshape(CHUNK, CHUNK)).reshape(-1)
    out4d = _gather_kernel(paddr, flat)
    return out4d.transpose(1, 3, 0, 2).reshape(BATCH, DIM)
